# Initial kernel scaffold; baseline (speedup 1.0000x reference)
import jax
import jax.numpy as jnp
from jax.experimental import pallas as pl


def kernel(images, features, proposals, W1, b1, W2, b2, Wc, bc, Wb, bb):
    raise NotImplementedError("write your pallas kernel here")


# speedup vs baseline: 5.7834x; 5.7834x over previous
import functools
import numpy as np
import jax
import jax.numpy as jnp
from jax import lax
from jax.experimental import pallas as pl
from jax.experimental.pallas import tpu as pltpu
from jax.experimental.pallas import tpu_sc as plsc

_NUM_CLASSES = 81
_POOL = 7
_SCALE = 1.0 / 16.0
_NMS_THRESH = 0.5
_SCORE_THRESH = 0.001
_DET = 100
_MIN_SIZE = 0.01
_LIM = float(np.log(1000.0 / 16.0))

_N = 2
_P = 1000
_PPAD = 1024
_NP = _N * _PPAD
_S = _POOL * _POOL
_NPTS = _NP * _S
_C = 256
_HW = 50
_NW = 32
_PW = _NPTS // _NW
_CH = 32
_NCH = _PW // _CH
_BP = 128
_NBLK = _NP // _BP


def _k0_body(prop_ref, i00, i01, i10, i11, w00, w01, w10, w11):
    x1 = prop_ref[:, 0:1] * _SCALE
    y1 = prop_ref[:, 1:2] * _SCALE
    x2 = prop_ref[:, 2:3] * _SCALE
    y2 = prop_ref[:, 3:4] * _SCALE
    bw = jnp.maximum(x2 - x1, 1e-3)
    bh = jnp.maximum(y2 - y1, 1e-3)
    s = lax.broadcasted_iota(jnp.int32, (_NP, _S), 1)
    gj = (s % _POOL).astype(jnp.float32)
    gi = (s // _POOL).astype(jnp.float32)
    gx = (gj + 0.5) / _POOL
    gy = (gi + 0.5) / _POOL
    px = x1 + bw * gx
    py = y1 + bh * gy
    x0f = jnp.floor(px)
    y0f = jnp.floor(py)
    wx = px - x0f
    wy = py - y0f
    x0 = jnp.clip(x0f.astype(jnp.int32), 0, _HW - 1)
    x1i = jnp.clip(x0 + 1, 0, _HW - 1)
    y0 = jnp.clip(y0f.astype(jnp.int32), 0, _HW - 1)
    y1i = jnp.clip(y0 + 1, 0, _HW - 1)
    prow = lax.broadcasted_iota(jnp.int32, (_NP, _S), 0)
    base = (prow // _PPAD) * (_HW * _HW)
    valid = (prow % _PPAD) < _P
    zi = jnp.zeros((_NP, _S), jnp.int32)
    zf = jnp.zeros((_NP, _S), jnp.float32)
    i00[...] = jnp.where(valid, base + y0 * _HW + x0, zi)
    i01[...] = jnp.where(valid, base + y0 * _HW + x1i, zi)
    i10[...] = jnp.where(valid, base + y1i * _HW + x0, zi)
    i11[...] = jnp.where(valid, base + y1i * _HW + x1i, zi)
    w00[...] = jnp.where(valid, (1.0 - wx) * (1.0 - wy), zf)
    w01[...] = jnp.where(valid, wx * (1.0 - wy), zf)
    w10[...] = jnp.where(valid, (1.0 - wx) * wy, zf)
    w11[...] = jnp.where(valid, wx * wy, zf)


def _k0(prop_pad, interpret=False):
    outs = [jax.ShapeDtypeStruct((_NP, _S), jnp.int32)] * 4 + \
           [jax.ShapeDtypeStruct((_NP, _S), jnp.float32)] * 4
    return pl.pallas_call(_k0_body, out_shape=outs, interpret=interpret)(prop_pad)


def _sc_gather(table, idx_sc, w_sc):
    mesh = plsc.VectorSubcoreMesh(core_axis_name="c", subcore_axis_name="s")

    @functools.partial(
        pl.kernel, mesh=mesh,
        out_type=jax.ShapeDtypeStruct((_NPTS, _C), jnp.float32),
        scratch_types=[
            pltpu.VMEM((_NCH * 4, _CH), jnp.int32),
            pltpu.VMEM((4, _CH, 16), jnp.float32),
            pltpu.VMEM((_CH, _C), jnp.float32),
            pltpu.VMEM((_CH, _C), jnp.float32),
            pltpu.VMEM((_CH, _C), jnp.float32),
            pltpu.VMEM((_CH, _C), jnp.float32),
            pltpu.VMEM((_CH, _C), jnp.float32),
            pltpu.SemaphoreType.DMA,
        ],
    )
    def k(table_hbm, idx_hbm, w_hbm, out_hbm,
          idx_v, w_ch, r00, r01, r10, r11, acc, sem):
        wid = lax.axis_index("s") * 2 + lax.axis_index("c")
        pltpu.sync_copy(idx_hbm.at[wid], idx_v)

        def chunk(g, _):
            j = g * 4
            h0 = pltpu.async_copy(table_hbm.at[idx_v.at[j]], r00, sem)
            h1 = pltpu.async_copy(table_hbm.at[idx_v.at[j + 1]], r01, sem)
            h2 = pltpu.async_copy(table_hbm.at[idx_v.at[j + 2]], r10, sem)
            h3 = pltpu.async_copy(table_hbm.at[idx_v.at[j + 3]], r11, sem)
            pltpu.sync_copy(w_hbm.at[wid, pl.ds(j, 4)], w_ch)
            h0.wait(); h1.wait(); h2.wait(); h3.wait()

            def point(p, _):
                a = w_ch[0, p, :]
                b = w_ch[1, p, :]
                c = w_ch[2, p, :]
                d = w_ch[3, p, :]
                for t in range(_C // 16):
                    sl = pl.ds(t * 16, 16)
                    top = a * r00[p, sl] + b * r01[p, sl]
                    bot = a * r10[p, sl] + b * r11[p, sl]
                    acc[p, sl] = top * c + bot * d
                return 0

            lax.fori_loop(0, _CH, point, 0)
            pltpu.sync_copy(acc, out_hbm.at[pl.ds(wid * _PW + g * _CH, _CH)])
            return 0

        lax.fori_loop(0, _NCH, chunk, 0)

    return k(table, idx_sc, w_sc)


def _k2_body(x_ref, prop_ref, w1_ref, b1_ref, w2_ref, b2_ref,
             wc_ref, bc_ref, wb_ref, bb_ref,
             sw_ref, lab_ref, det_ref):
    pid = pl.program_id(0)
    x = x_ref[...]
    h = jnp.maximum(jnp.dot(x, w1_ref[...],
                            preferred_element_type=jnp.float32)
                    + b1_ref[...], 0.0)
    h = jnp.maximum(jnp.dot(h, w2_ref[...],
                            preferred_element_type=jnp.float32)
                    + b2_ref[...], 0.0)
    logits = jnp.dot(h, wc_ref[...],
                     preferred_element_type=jnp.float32) + bc_ref[...]
    deltas = jnp.dot(h, wb_ref[...],
                     preferred_element_type=jnp.float32) + bb_ref[...]

    lmax = jnp.max(logits, axis=1, keepdims=True)
    e = jnp.exp(logits - lmax)
    probs = e / jnp.sum(e, axis=1, keepdims=True)
    lane = lax.broadcasted_iota(jnp.int32, (_BP, _NUM_CLASSES), 1)
    pmask = jnp.where(lane >= 1, probs, -1.0)
    m = jnp.max(pmask, axis=1, keepdims=True)
    labels = jnp.min(jnp.where(pmask == m, lane, 10 ** 6), axis=1,
                     keepdims=True)
    scores = m

    col = lax.broadcasted_iota(jnp.int32, (_BP, _NUM_CLASSES * 4), 1)
    cmask = (col // 4) == labels
    md = jnp.where(cmask, deltas, 0.0)
    k4 = col % 4
    d0 = jnp.sum(jnp.where(k4 == 0, md, 0.0), axis=1, keepdims=True)
    d1 = jnp.sum(jnp.where(k4 == 1, md, 0.0), axis=1, keepdims=True)
    d2 = jnp.sum(jnp.where(k4 == 2, md, 0.0), axis=1, keepdims=True)
    d3 = jnp.sum(jnp.where(k4 == 3, md, 0.0), axis=1, keepdims=True)

    px1 = prop_ref[:, 0:1]
    py1 = prop_ref[:, 1:2]
    px2 = prop_ref[:, 2:3]
    py2 = prop_ref[:, 3:4]
    widths = px2 - px1
    heights = py2 - py1
    cx = px1 + 0.5 * widths
    cy = py1 + 0.5 * heights
    dx = d0 / 10.0
    dy = d1 / 10.0
    dw = jnp.minimum(d2 / 5.0, _LIM)
    dh = jnp.minimum(d3 / 5.0, _LIM)
    pcx = dx * widths + cx
    pcy = dy * heights + cy
    pw = jnp.exp(dw) * widths
    ph = jnp.exp(dh) * heights
    bx1 = jnp.clip(pcx - 0.5 * pw, 0.0, 800.0)
    by1 = jnp.clip(pcy - 0.5 * ph, 0.0, 800.0)
    bx2 = jnp.clip(pcx + 0.5 * pw, 0.0, 800.0)
    by2 = jnp.clip(pcy + 0.5 * ph, 0.0, 800.0)

    grow = pid * _BP + lax.broadcasted_iota(jnp.int32, (_BP, 1), 0)
    vrow = (grow % _PPAD) < _P
    valid = ((bx2 - bx1 >= _MIN_SIZE) & (by2 - by1 >= _MIN_SIZE)
             & (scores > _SCORE_THRESH) & vrow)
    sw = jnp.where(valid, scores, -1e9)

    sw_ref[0, 0, :] = sw[:, 0]
    lab_ref[0, 0, :] = labels[:, 0].astype(jnp.float32)
    det_ref[0, 0, :] = bx1[:, 0]
    det_ref[0, 1, :] = by1[:, 0]
    det_ref[0, 2, :] = bx2[:, 0]
    det_ref[0, 3, :] = by2[:, 0]


def _k2(x_mlp, prop_pad, W1p, b1, W2, b2, Wc, bc, Wb, bb, interpret=False):
    outs = [jax.ShapeDtypeStruct((_NBLK, 1, _BP), jnp.float32),
            jax.ShapeDtypeStruct((_NBLK, 1, _BP), jnp.float32),
            jax.ShapeDtypeStruct((_NBLK, 4, _BP), jnp.float32)]
    full = lambda shape: pl.BlockSpec(shape, lambda i: (0, 0))
    return pl.pallas_call(
        _k2_body,
        grid=(_NBLK,),
        in_specs=[
            pl.BlockSpec((_BP, _S * _C), lambda i: (i, 0)),
            pl.BlockSpec((_BP, 4), lambda i: (i, 0)),
            full((_S * _C, 512)),
            full((1, 512)),
            full((512, 512)),
            full((1, 512)),
            full((512, _NUM_CLASSES)),
            full((1, _NUM_CLASSES)),
            full((512, _NUM_CLASSES * 4)),
            full((1, _NUM_CLASSES * 4)),
        ],
        out_specs=[
            pl.BlockSpec((1, 1, _BP), lambda i: (i, 0, 0)),
            pl.BlockSpec((1, 1, _BP), lambda i: (i, 0, 0)),
            pl.BlockSpec((1, 4, _BP), lambda i: (i, 0, 0)),
        ],
        out_shape=outs,
        interpret=interpret,
    )(x_mlp, prop_pad, W1p, b1, W2, b2, Wc, bc, Wb, bb)


def _k3_body(sw_ref, lab_ref, x1_ref, y1_ref, x2_ref, y2_ref,
             os_ref, ol_ref, ox1_ref, oy1_ref, ox2_ref, oy2_ref):
    sw = sw_ref[0]
    labf = lab_ref[0]
    rx1 = x1_ref[0]
    ry1 = y1_ref[0]
    rx2 = x2_ref[0]
    ry2 = y2_ref[0]
    off = labf * 10000.0
    bx1 = rx1 + off
    by1 = ry1 + off
    bx2 = rx2 + off
    by2 = ry2 + off
    area = (bx2 - bx1) * (by2 - by1)
    fi = (lax.broadcasted_iota(jnp.int32, (8, _BP), 0) * _BP
          + lax.broadcasted_iota(jnp.int32, (8, _BP), 1))
    lane = lax.broadcasted_iota(jnp.int32, (1, _BP), 1)
    zrow = jnp.zeros((1, _BP), jnp.float32)

    def step(t, carry):
        s_cur, o_s, o_l, o_x1, o_y1, o_x2, o_y2 = carry
        mval = jnp.max(s_cur)
        idx = jnp.min(jnp.where(s_cur == mval, fi, 2 ** 30))
        sel = fi == idx
        pick = lambda v: jnp.sum(jnp.where(sel, v, 0.0))
        sx1 = pick(bx1)
        sy1 = pick(by1)
        sx2 = pick(bx2)
        sy2 = pick(by2)
        sarea = pick(area)
        s_orig_v = pick(sw)
        s_lab = pick(labf)
        hit = lane == t
        o_s = jnp.where(hit, s_orig_v, o_s)
        o_l = jnp.where(hit, jnp.where(s_orig_v > -1e8, s_lab, -1.0), o_l)
        o_x1 = jnp.where(hit, pick(rx1), o_x1)
        o_y1 = jnp.where(hit, pick(ry1), o_y1)
        o_x2 = jnp.where(hit, pick(rx2), o_x2)
        o_y2 = jnp.where(hit, pick(ry2), o_y2)
        ix1 = jnp.maximum(sx1, bx1)
        iy1 = jnp.maximum(sy1, by1)
        ix2 = jnp.minimum(sx2, bx2)
        iy2 = jnp.minimum(sy2, by2)
        inter = jnp.maximum(ix2 - ix1, 0.0) * jnp.maximum(iy2 - iy1, 0.0)
        iou = inter / (sarea + area - inter + 1e-9)
        s_cur = jnp.where(iou > _NMS_THRESH, -1e9, s_cur)
        s_cur = jnp.where(sel, -1e9, s_cur)
        return s_cur, o_s, o_l, o_x1, o_y1, o_x2, o_y2

    init = (sw, zrow, zrow, zrow, zrow, zrow, zrow)
    _, o_s, o_l, o_x1, o_y1, o_x2, o_y2 = lax.fori_loop(0, _DET, step, init)
    os_ref[0] = o_s
    ol_ref[0] = o_l
    ox1_ref[0] = o_x1
    oy1_ref[0] = o_y1
    ox2_ref[0] = o_x2
    oy2_ref[0] = o_y2


def _k3(sw, labf, dx1, dy1, dx2, dy2, interpret=False):
    outs = [jax.ShapeDtypeStruct((_N, 1, _BP), jnp.float32)] * 6
    spec_in = pl.BlockSpec((1, 8, _BP), lambda i: (i, 0, 0))
    spec_out = pl.BlockSpec((1, 1, _BP), lambda i: (i, 0, 0))
    return pl.pallas_call(
        _k3_body,
        grid=(_N,),
        in_specs=[spec_in] * 6,
        out_specs=[spec_out] * 6,
        out_shape=outs,
        interpret=interpret,
    )(sw, labf, dx1, dy1, dx2, dy2)


def kernel(images, features, proposals, W1, b1, W2, b2, Wc, bc, Wb, bb):
    del images
    prop_pad = jnp.zeros((_N, _PPAD, 4), jnp.float32)
    prop_pad = prop_pad.at[:, :_P, :].set(proposals).reshape(_NP, 4)
    table = jnp.transpose(features, (0, 2, 3, 1)).reshape(_N * _HW * _HW, _C)
    W1p = W1.reshape(_C, _S, 512).transpose(1, 0, 2).reshape(_S * _C, 512)

    x1 = proposals[:, :, 0] * _SCALE
    y1 = proposals[:, :, 1] * _SCALE
    x2 = proposals[:, :, 2] * _SCALE
    y2 = proposals[:, :, 3] * _SCALE
    bw = jnp.maximum(x2 - x1, 1e-3)
    bh = jnp.maximum(y2 - y1, 1e-3)
    g = (jnp.arange(_POOL, dtype=jnp.float32) + 0.5) / _POOL
    px = x1[:, :, None] + bw[:, :, None] * g[None, None, :]
    py = y1[:, :, None] + bh[:, :, None] * g[None, None, :]
    xx = jnp.broadcast_to(px[:, :, None, :], (_N, _P, _POOL, _POOL))
    yy = jnp.broadcast_to(py[:, :, :, None], (_N, _P, _POOL, _POOL))
    x0f = jnp.floor(xx)
    y0f = jnp.floor(yy)
    wx = xx - x0f
    wy = yy - y0f
    x0 = jnp.clip(x0f.astype(jnp.int32), 0, _HW - 1)
    x1i = jnp.clip(x0 + 1, 0, _HW - 1)
    y0 = jnp.clip(y0f.astype(jnp.int32), 0, _HW - 1)
    y1i = jnp.clip(y0 + 1, 0, _HW - 1)
    base = (jnp.arange(_N, dtype=jnp.int32) * (_HW * _HW))[:, None, None, None]

    def padi(a):
        out = jnp.zeros((_N, _PPAD, _S), jnp.int32)
        return out.at[:, :_P, :].set(a.reshape(_N, _P, _S)).reshape(_NP, _S)

    def padf(a):
        out = jnp.zeros((_N, _PPAD, _S), jnp.float32)
        return out.at[:, :_P, :].set(a.reshape(_N, _P, _S)).reshape(_NP, _S)

    i00 = padi(base + y0 * _HW + x0)
    i01 = padi(base + y0 * _HW + x1i)
    i10 = padi(base + y1i * _HW + x0)
    i11 = padi(base + y1i * _HW + x1i)
    w00 = padf(1.0 - wx)
    w01 = padf(wx)
    w10 = padf(1.0 - wy)
    w11 = padf(wy)

    def shuf(a):
        return a.reshape(_NW, _NCH, _CH)

    idx_sc = jnp.stack([shuf(i00), shuf(i01), shuf(i10), shuf(i11)],
                       axis=2).reshape(_NW, _NCH * 4, _CH)
    w_sc = jnp.stack([shuf(w00), shuf(w01), shuf(w10), shuf(w11)],
                     axis=2).reshape(_NW, _NCH * 4, _CH)
    w_sc = jnp.broadcast_to(w_sc[..., None], (_NW, _NCH * 4, _CH, 16)) + 0.0

    x = _sc_gather(table, idx_sc, w_sc)
    x_mlp = x.reshape(_NP, _S * _C)

    sw3, lab3, det3 = _k2(x_mlp, prop_pad, W1p,
                          b1.reshape(1, 512), W2, b2.reshape(1, 512),
                          Wc, bc.reshape(1, _NUM_CLASSES),
                          Wb, bb.reshape(1, _NUM_CLASSES * 4))

    sw = sw3.reshape(_N, 8, _BP)
    labf = lab3.reshape(_N, 8, _BP)
    det = det3.reshape(_NBLK, 4, _BP)
    dx1 = det[:, 0, :].reshape(_N, 8, _BP)
    dy1 = det[:, 1, :].reshape(_N, 8, _BP)
    dx2 = det[:, 2, :].reshape(_N, 8, _BP)
    dy2 = det[:, 3, :].reshape(_N, 8, _BP)

    o_s, o_l, o_x1, o_y1, o_x2, o_y2 = _k3(sw, labf, dx1, dy1, dx2, dy2)

    out_s = o_s[:, 0, :_DET]
    out_l = o_l[:, 0, :_DET].astype(jnp.int32)
    out_d = jnp.stack([o_x1[:, 0, :_DET], o_y1[:, 0, :_DET],
                       o_x2[:, 0, :_DET], o_y2[:, 0, :_DET]], axis=-1)
    return out_l, out_s, out_d
